# baseline (speedup 1.0000x reference)
import functools

import jax
import jax.numpy as jnp
from jax.experimental import pallas as pl
from jax.experimental.pallas import tpu as pltpu

_ROWS = 512
_TCHUNK = 8


def _ceil_to(n, m):
    return ((n + m - 1) // m) * m


def _lstm_pair(places_ref, x_ref, wih_ref, whh_ref, b_ref, out_ref, gin_ref,
               *, hidden):
    S, Bt, D = x_ref.shape
    H = hidden
    C = gin_ref.shape[0] // Bt
    half = Bt // 2

    whh = whh_ref[...]
    bias = b_ref[...]
    pl_a = places_ref[0:half]
    pl_b = places_ref[half:Bt]

    def make_state():
        z = jnp.zeros((half, H), jnp.float32)
        return z, z, z

    st_a = make_state()
    st_b = make_state()

    def step(t, row, places, st):
        h, c, out = st
        gates = (jnp.dot(h, whh, preferred_element_type=jnp.float32)
                 + gin_ref[pl.ds(row, half), :])
        act = jax.nn.sigmoid(gates[:, 0:3 * H])
        i_g = act[:, 0 * H:1 * H]
        f_g = act[:, 1 * H:2 * H]
        o_g = act[:, 2 * H:3 * H]
        g_g = jnp.tanh(gates[:, 3 * H:4 * H])
        c = f_g * c + i_g * g_g
        h = o_g * jnp.tanh(c)
        out = jnp.where(places == t, h, out)
        return h, c, out

    def body(j, carry, base):
        st_a, st_b = carry
        r = pl.multiple_of(j * Bt, Bt)
        st_a = step(base + j, r, pl_a, st_a)
        st_b = step(base + j, r + half, pl_b, st_b)
        return st_a, st_b

    for k in range(S // C):
        xc = x_ref[k * C:(k + 1) * C].reshape(C * Bt, D)
        gin_ref[...] = (
            jnp.dot(xc, wih_ref[...], preferred_element_type=jnp.float32)
            + bias)
        st_a, st_b = jax.lax.fori_loop(
            0, C, functools.partial(body, base=k * C), (st_a, st_b),
            unroll=2)

    out_ref[0:half] = st_a[2]
    out_ref[half:Bt] = st_b[2]


@jax.jit
def kernel(x, real_positions, wih_packed, whh_packed, bias_packed):
    B, S, D = x.shape
    H, Gp = whh_packed.shape
    Bt = _ROWS
    Bp = _ceil_to(B, Bt)
    C = _TCHUNK if S % _TCHUNK == 0 else S

    x_tm = jnp.transpose(x.astype(jnp.bfloat16), (1, 0, 2))
    if Bp != B:
        x_tm = jnp.pad(x_tm, ((0, 0), (0, Bp - B), (0, 0)))

    lengths = jnp.sum(real_positions.astype(jnp.float32), axis=-1)
    places = lengths.astype(jnp.int32) - 1
    places = jnp.where(places < 0, places + S, places)[:, None]
    if Bp != B:
        places = jnp.pad(places, ((0, Bp - B), (0, 0)))

    out = pl.pallas_call(
        functools.partial(_lstm_pair, hidden=H),
        out_shape=jax.ShapeDtypeStruct((Bp, H), jnp.float32),
        grid_spec=pltpu.PrefetchScalarGridSpec(
            num_scalar_prefetch=0,
            grid=(Bp // Bt,),
            in_specs=[
                pl.BlockSpec((Bt, 1), lambda g: (g, 0)),
                pl.BlockSpec((S, Bt, D), lambda g: (0, g, 0)),
                pl.BlockSpec((D, Gp), lambda g: (0, 0)),
                pl.BlockSpec((H, Gp), lambda g: (0, 0)),
                pl.BlockSpec((1, Gp), lambda g: (0, 0)),
            ],
            out_specs=pl.BlockSpec((Bt, H), lambda g: (g, 0)),
            scratch_shapes=[pltpu.VMEM((C * Bt, Gp), jnp.float32)],
        ),
        compiler_params=pltpu.CompilerParams(
            dimension_semantics=("parallel",)),
    )(places, x_tm, wih_packed, whh_packed, bias_packed)

    return out[:B]


# speedup vs baseline: 1.1546x; 1.1546x over previous
import functools

import jax
import jax.numpy as jnp
from jax.experimental import pallas as pl
from jax.experimental.pallas import tpu as pltpu

_ROWS = 256
_TCHUNK = 16


def _ceil_to(n, m):
    return ((n + m - 1) // m) * m


def _lstm_tile(places_ref, x_ref, wih_ref, whh_ref, b_ref, out_ref, gin_ref,
               *, hidden):
    S, Bt, D = x_ref.shape
    H = hidden
    C = gin_ref.shape[0] // Bt
    half = Bt // 2

    whh = whh_ref[...]
    bias = b_ref[...]
    pl_a = places_ref[0:half]
    pl_b = places_ref[half:Bt]

    def make_state():
        z = jnp.zeros((half, H), jnp.float32)
        return z, z, z

    st_a = make_state()
    st_b = make_state()

    def step(t, row, places, st):
        h, c, out = st
        gates = (jnp.dot(h, whh, preferred_element_type=jnp.float32)
                 + gin_ref[pl.ds(row, half), :])
        act = jax.nn.sigmoid(gates[:, 0:3 * H])
        i_g = act[:, 0 * H:1 * H]
        f_g = act[:, 1 * H:2 * H]
        o_g = act[:, 2 * H:3 * H]
        g_g = jnp.tanh(gates[:, 3 * H:4 * H])
        c = f_g * c + i_g * g_g
        h = o_g * jnp.tanh(c)
        out = jnp.where(places == t, h, out)
        return h, c, out

    for k in range(S // C):
        xc = x_ref[k * C:(k + 1) * C].reshape(C * Bt, D)
        gin_ref[...] = (
            jnp.dot(xc, wih_ref[...], preferred_element_type=jnp.float32)
            + bias)
        for j in range(C):
            r = pl.multiple_of(j * Bt, Bt)
            st_a = step(k * C + j, r, pl_a, st_a)
            st_b = step(k * C + j, r + half, pl_b, st_b)

    out_ref[0:half] = st_a[2]
    out_ref[half:Bt] = st_b[2]


@jax.jit
def kernel(x, real_positions, wih_packed, whh_packed, bias_packed):
    B, S, D = x.shape
    H, Gp = whh_packed.shape
    Bt = _ROWS
    Bp = _ceil_to(B, Bt)
    C = _TCHUNK if S % _TCHUNK == 0 else S

    x_tm = jnp.transpose(x.astype(jnp.bfloat16), (1, 0, 2))
    if Bp != B:
        x_tm = jnp.pad(x_tm, ((0, 0), (0, Bp - B), (0, 0)))

    lengths = jnp.sum(real_positions.astype(jnp.float32), axis=-1)
    places = lengths.astype(jnp.int32) - 1
    places = jnp.where(places < 0, places + S, places)[:, None]
    if Bp != B:
        places = jnp.pad(places, ((0, Bp - B), (0, 0)))

    out = pl.pallas_call(
        functools.partial(_lstm_tile, hidden=H),
        out_shape=jax.ShapeDtypeStruct((Bp, H), jnp.float32),
        grid_spec=pltpu.PrefetchScalarGridSpec(
            num_scalar_prefetch=0,
            grid=(Bp // Bt,),
            in_specs=[
                pl.BlockSpec((Bt, 1), lambda g: (g, 0)),
                pl.BlockSpec((S, Bt, D), lambda g: (0, g, 0)),
                pl.BlockSpec((D, Gp), lambda g: (0, 0)),
                pl.BlockSpec((H, Gp), lambda g: (0, 0)),
                pl.BlockSpec((1, Gp), lambda g: (0, 0)),
            ],
            out_specs=pl.BlockSpec((Bt, H), lambda g: (g, 0)),
            scratch_shapes=[pltpu.VMEM((C * Bt, Gp), jnp.float32)],
        ),
        compiler_params=pltpu.CompilerParams(
            dimension_semantics=("parallel",)),
    )(places, x_tm, wih_packed, whh_packed, bias_packed)

    return out[:B]
